# Initial kernel scaffold; baseline (speedup 1.0000x reference)
import jax
import jax.numpy as jnp
from jax.experimental import pallas as pl


def kernel(x, edge_index1, edge_weight1, edge_index2, edge_weight2, b, W1, b1, W2, b2, Wfc, bfc):
    raise NotImplementedError("write your pallas kernel here")


# speedup vs baseline: 27.7855x; 27.7855x over previous
import functools

import jax
import jax.numpy as jnp
from jax import lax
from jax.experimental import pallas as pl
from jax.experimental.pallas import tpu as pltpu
from jax.experimental.pallas import tpu_sc as plsc

N1, N2, T, B = 10000, 1000, 15, 16
G1, G2, K1, K2, C = 64, 32, 12, 12, 6

NW = 32
F32 = jnp.float32
I32 = jnp.int32

_GDN = lax.GatherDimensionNumbers(
    offset_dims=(), collapsed_slice_dims=(0,), start_index_map=(0,))


def _bcast_lane(v, e):
    return lax.gather(v, jnp.full((16, 1), e, I32), _GDN, (1,),
                      mode=lax.GatherScatterMode.PROMISE_IN_BOUNDS)


def _build_spmv(Np, Wt, KE, alpha, beta, name):
    rows_pt = Np // NW
    nvec = Wt // 16
    n_rblk = rows_pt // KE

    mesh = plsc.VectorSubcoreMesh(core_axis_name="c", subcore_axis_name="s")

    @functools.partial(
        pl.kernel,
        out_type=jax.ShapeDtypeStruct((Np, Wt), F32),
        mesh=mesh,
        compiler_params=pltpu.CompilerParams(needs_layout_passes=False),
        scratch_types=[
            pltpu.MemorySpace.VMEM((rows_pt, Wt), F32),
            pltpu.MemorySpace.VMEM((2, KE, Wt), F32),
            pltpu.MemorySpace.VMEM((2, KE), I32),
            pltpu.MemorySpace.VMEM((2, KE), I32),
            pltpu.MemorySpace.VMEM((2, KE), F32),
            pltpu.MemorySpace.VMEM((KE,), F32),
            pltpu.MemorySpace.VMEM((KE,), I32),
            pltpu.MemorySpace.VMEM((2, NW), I32),
            pltpu.SemaphoreType.DMA,
            pltpu.SemaphoreType.DMA,
        ],
        name=name,
    )
    def step(src_h, dst_h, w_h, eb_h, xprev_h, *rest):
        if beta != 0.0:
            xm2_h, out_h, acc, rows_d, src_d, dst_d, w_d, wm_v, dl_v, ebv, \
                gsem0, gsem1 = rest
        else:
            xm2_h = None
            out_h, acc, rows_d, src_d, dst_d, w_d, wm_v, dl_v, ebv, \
                gsem0, gsem1 = rest
        gsems = (gsem0, gsem1)
        cid = lax.axis_index("c")
        sid = lax.axis_index("s")
        wid = cid * 16 + sid
        row0 = wid * rows_pt
        iot = lax.iota(I32, 16)

        pltpu.sync_copy(eb_h, ebv)
        sel = iot == sid
        s_lo = jnp.sum(jnp.where(sel, ebv[0, pl.ds(cid * 16, 16)], 0))
        s_hi = jnp.sum(jnp.where(sel, ebv[1, pl.ds(cid * 16, 16)], 0))
        off0 = (s_lo // 8) * 8
        nch = (s_hi - off0 + KE - 1) // KE

        def zrow(r, _):
            for j in range(nvec):
                acc[r, pl.ds(j * 16, 16)] = jnp.zeros((16,), F32)
            return 0
        lax.fori_loop(0, rows_pt, zrow, 0)

        def start(i, par):
            off = off0 + i * KE
            pltpu.sync_copy(src_h.at[pl.ds(off, KE)], src_d.at[par])
            pltpu.sync_copy(dst_h.at[pl.ds(off, KE)], dst_d.at[par])
            pltpu.sync_copy(w_h.at[pl.ds(off, KE)], w_d.at[par])
            pltpu.async_copy(xprev_h.at[src_d.at[par]], rows_d.at[par],
                             gsems[par])

        def process(i, par):
            off = off0 + i * KE
            for g in range(KE // 16):
                sl = pl.ds(g * 16, 16)
                ids = off + g * 16 + iot
                keep = (ids >= s_lo) & (ids < s_hi)
                wm_v[sl] = jnp.where(keep, w_d[par, sl], 0.0)
                dl = dst_d[par, sl] - row0
                dl_v[sl] = jnp.clip(dl, 0, rows_pt - 1)

            def edge_work(g, e, w16, d16):
                ws = _bcast_lane(w16, e)
                db = _bcast_lane(d16, e)
                r = g * 16 + e
                for j in range(nvec):
                    jsl = pl.ds(j * 16, 16)
                    plsc.addupdate_scatter(
                        acc, [db, j * 16 + iot], rows_d[par, r, jsl] * ws)

            def group(g, _):
                gsl = pl.ds(g * 16, 16)
                w16 = wm_v[gsl]
                d16 = dl_v[gsl]
                if nvec <= 16:
                    for e in range(16):
                        edge_work(g, e, w16, d16)
                else:
                    def edge(e, _):
                        edge_work(g, e, w16, d16)
                        return 0
                    lax.fori_loop(0, 16, edge, 0)
                return 0
            lax.fori_loop(0, KE // 16, group, 0)

        def stage(i, par):
            @pl.when(i < nch)
            def _():
                @pl.when(i + 1 < nch)
                def _():
                    start(i + 1, 1 - par)
                pltpu.make_async_copy(
                    xprev_h.at[src_d.at[par]], rows_d.at[par],
                    gsems[par]).wait()
                process(i, par)

        @pl.when(nch > 0)
        def _():
            start(0, 0)

        def pair(p, _):
            stage(p * 2, 0)
            stage(p * 2 + 1, 1)
            return 0
        lax.fori_loop(0, (nch + 1) // 2, pair, 0)

        for rb in range(n_rblk):
            ra = rb * KE
            if beta != 0.0:
                pltpu.sync_copy(xm2_h.at[pl.ds(row0 + ra, KE)], rows_d.at[0])

            def prow(r, _):
                for j in range(nvec):
                    jsl = pl.ds(j * 16, 16)
                    v = acc[ra + r, jsl] * alpha
                    if beta != 0.0:
                        v = v + rows_d[0, r, jsl] * beta
                    acc[ra + r, jsl] = v
                return 0
            lax.fori_loop(0, KE, prow, 0)
            pltpu.sync_copy(acc.at[pl.ds(ra, KE)],
                            out_h.at[pl.ds(row0 + ra, KE)])

    return step


def _cheb_contract(K, Np, Wt, Fout, BN, relu, name):
    nblk = Np // BN

    def body(*refs):
        z_refs, wt_ref, bias_ref, o_ref = refs[:K], refs[K], refs[K + 1], \
            refs[K + 2]
        acc = jnp.zeros((BN, Fout), F32)
        for k in range(K):
            acc = acc + jnp.dot(z_refs[k][...], wt_ref[k],
                                preferred_element_type=F32)
        acc = acc + bias_ref[...]
        if relu:
            acc = jnp.maximum(acc, 0.0)
        o_ref[...] = acc

    return pl.pallas_call(
        body,
        grid=(nblk,),
        in_specs=[pl.BlockSpec((BN, Wt), lambda i: (i, 0))] * K + [
            pl.BlockSpec((K, Wt, Fout), lambda i: (0, 0, 0)),
            pl.BlockSpec((1, Fout), lambda i: (0, 0)),
        ],
        out_specs=pl.BlockSpec((BN, Fout), lambda i: (i, 0)),
        out_shape=jax.ShapeDtypeStruct((Np, Fout), F32),
        name=name,
    )


def _coarsen(M, N, F, BK):
    nblk = N // BK

    def body(b_ref, h_ref, o_ref):
        @pl.when(pl.program_id(0) == 0)
        def _():
            o_ref[...] = jnp.zeros_like(o_ref)
        o_ref[...] += jnp.dot(b_ref[...], h_ref[...],
                              preferred_element_type=F32)

    return pl.pallas_call(
        body,
        grid=(nblk,),
        in_specs=[
            pl.BlockSpec((M, BK), lambda j: (0, j)),
            pl.BlockSpec((BK, F), lambda j: (j, 0)),
        ],
        out_specs=pl.BlockSpec((M, F), lambda j: (0, 0)),
        out_shape=jax.ShapeDtypeStruct((M, F), F32),
        name="coarsen",
    )


def _fc_head(Bn, Kd, Cn):
    def body(h_ref, wt_ref, b_ref, o_ref):
        logits = lax.dot_general(
            h_ref[...], wt_ref[...], (((1,), (1,)), ((), ())),
            preferred_element_type=F32)
        logits = logits + b_ref[...]
        m = jnp.max(logits, axis=1, keepdims=True)
        z = logits - m
        lse = jnp.log(jnp.sum(jnp.exp(z), axis=1, keepdims=True))
        o_ref[...] = z - lse

    return pl.pallas_call(
        body,
        in_specs=[
            pl.BlockSpec((Bn, Kd), lambda: (0, 0)),
            pl.BlockSpec((Cn, Kd), lambda: (0, 0)),
            pl.BlockSpec((1, Cn), lambda: (0, 0)),
        ],
        out_specs=pl.BlockSpec((Bn, Cn), lambda: (0, 0)),
        out_shape=jax.ShapeDtypeStruct((Bn, Cn), F32),
        name="fc_head",
    )


def _sort_edges(src, dst, w, Np, KE):
    E = src.shape[0]
    L = (E + KE - 1) // KE * KE + KE
    pad = L - E
    srcp = jnp.concatenate([src, jnp.zeros((pad,), I32)])
    dstp = jnp.concatenate([dst, jnp.full((pad,), Np, I32)])
    wp = jnp.concatenate([w, jnp.zeros((pad,), F32)])
    order = jnp.argsort(dstp)
    srcs, dsts, ws = srcp[order], dstp[order], wp[order]
    rows_pt = Np // NW
    lo = jnp.searchsorted(dsts, jnp.arange(NW, dtype=I32) * rows_pt)
    hi = jnp.searchsorted(dsts, (jnp.arange(NW, dtype=I32) + 1) * rows_pt)
    ebounds = jnp.stack([lo, hi]).astype(I32)
    return srcs, dsts, ws, ebounds


def _expand_weight(Wr, Fin, Fout):
    K = Wr.shape[0]
    eye = jnp.eye(B, dtype=F32)
    Wm = (Wr[:, :, None, :, None] * eye[None, None, :, None, :])
    return Wm.reshape(K, Fin * B, Fout * B)


def kernel(x, edge_index1, edge_weight1, edge_index2, edge_weight2, b,
           W1, b1, W2, b2, Wfc, bfc):
    Wt1c = 256
    N1p = 10240
    X0 = jnp.transpose(x, (1, 2, 0)).reshape(N1, T * B)
    X0 = jnp.pad(X0, ((0, N1p - N1), (0, Wt1c - T * B)))

    KE1 = 64
    src1, dst1, w1, eb1 = _sort_edges(edge_index1[0], edge_index1[1],
                                      edge_weight1, N1p, KE1)
    spmv1_a = _build_spmv(N1p, Wt1c, KE1, 1.0, 0.0, "sc_spmv1_first")
    spmv1_b = _build_spmv(N1p, Wt1c, KE1, 2.0, -1.0, "sc_spmv1_rec")

    Z = [X0]
    Z.append(spmv1_a(src1, dst1, w1, eb1, X0))
    for _ in range(2, K1):
        Z.append(spmv1_b(src1, dst1, w1, eb1, Z[-1], Z[-2]))

    Wm1 = _expand_weight(W1[:, :, 0, :], T, G1)
    Wm1 = jnp.pad(Wm1, ((0, 0), (0, Wt1c - T * B), (0, 0)))
    bias1 = jnp.repeat(b1, B)[None, :]
    H = _cheb_contract(K1, N1p, Wt1c, G1 * B, 512, True, "cheb1")(
        *Z, Wm1, bias1)

    bp = jnp.pad(b, ((0, 0), (0, N1p - N1)))
    Cm = _coarsen(N2, N1p, G1 * B, 1024)(bp, H)

    N2p = 1024
    Wt2c = G1 * B
    Cp = jnp.pad(Cm, ((0, N2p - N2), (0, 0)))

    KE2 = 32
    src2, dst2, w2, eb2 = _sort_edges(edge_index2[0], edge_index2[1],
                                      edge_weight2, N2p, KE2)
    spmv2_a = _build_spmv(N2p, Wt2c, KE2, 1.0, 0.0, "sc_spmv2_first")
    spmv2_b = _build_spmv(N2p, Wt2c, KE2, 2.0, -1.0, "sc_spmv2_rec")

    Z2 = [Cp]
    Z2.append(spmv2_a(src2, dst2, w2, eb2, Cp))
    for _ in range(2, K2):
        Z2.append(spmv2_b(src2, dst2, w2, eb2, Z2[-1], Z2[-2]))

    Wm2 = _expand_weight(W2[:, 0, :, :], G1, G2)
    bias2 = jnp.repeat(b2, B)[None, :]
    Y = _cheb_contract(K2, N2p, Wt2c, G2 * B, 128, False, "cheb2")(
        *Z2, Wm2, bias2)

    h2d = Y[:N2].reshape(B, N2 * G2)
    return _fc_head(B, N2 * G2, C)(h2d, Wfc.T, bfc[None, :])


# speedup vs baseline: 62.7963x; 2.2600x over previous
import functools

import jax
import jax.numpy as jnp
from jax import lax
from jax.experimental import pallas as pl
from jax.experimental.pallas import tpu as pltpu
from jax.experimental.pallas import tpu_sc as plsc

N1, N2, T, B = 10000, 1000, 15, 16
G1, G2, K1, K2, C = 64, 32, 12, 12, 6

NW = 32
F32 = jnp.float32
I32 = jnp.int32

_GDN = lax.GatherDimensionNumbers(
    offset_dims=(), collapsed_slice_dims=(0,), start_index_map=(0,))


def _bcast_lane(v, e):
    return lax.gather(v, jnp.full((16, 1), e, I32), _GDN, (1,),
                      mode=lax.GatherScatterMode.PROMISE_IN_BOUNDS)


def _build_spmv(Np, Wt, KE, alpha, beta, name):
    rows_pt = Np // NW
    nvec = Wt // 16
    n_rblk = rows_pt // KE
    reg_rows = nvec <= 16

    mesh = plsc.VectorSubcoreMesh(core_axis_name="c", subcore_axis_name="s")

    @functools.partial(
        pl.kernel,
        out_type=jax.ShapeDtypeStruct((Np, Wt), F32),
        mesh=mesh,
        compiler_params=pltpu.CompilerParams(needs_layout_passes=False),
        scratch_types=[
            pltpu.MemorySpace.VMEM((rows_pt, Wt), F32),
            pltpu.MemorySpace.VMEM((2, KE, Wt), F32),
            pltpu.MemorySpace.VMEM((2, KE), I32),
            pltpu.MemorySpace.VMEM((2, KE), F32),
            pltpu.MemorySpace.VMEM((rows_pt + 16,), I32),
            pltpu.SemaphoreType.DMA,
            pltpu.SemaphoreType.DMA,
        ],
        name=name,
    )
    def step(src_h, w_h, rp_h, xprev_h, *rest):
        if beta != 0.0:
            xm2_h, out_h, acc, rows_d, src_d, w_d, rp_v, gsem0, gsem1 = rest
        else:
            xm2_h = None
            out_h, acc, rows_d, src_d, w_d, rp_v, gsem0, gsem1 = rest
        cid = lax.axis_index("c")
        sid = lax.axis_index("s")
        wid = cid * 16 + sid
        row0 = wid * rows_pt
        iot = lax.iota(I32, 16)

        pltpu.sync_copy(rp_h.at[pl.ds(row0, rows_pt + 16)], rp_v)

        def extract(idx):
            base = (idx // 16) * 16
            part = rp_v[pl.ds(base, 16)]
            return jnp.sum(jnp.where(iot == idx - base, part, 0))

        e_lo = extract(0)
        e_hi = extract(rows_pt)
        off0 = (e_lo // KE) * KE
        nch = (e_hi - off0 + KE - 1) // KE

        if not reg_rows:
            def zrow(r, _):
                for j in range(nvec):
                    acc[r, pl.ds(j * 16, 16)] = jnp.zeros((16,), F32)
                return 0
            lax.fori_loop(0, rows_pt, zrow, 0)

        def start(i, par):
            off = off0 + i * KE
            pltpu.sync_copy(src_h.at[pl.ds(off, KE)], src_d.at[par])
            pltpu.sync_copy(w_h.at[pl.ds(off, KE)], w_d.at[par])
            sem = gsem0 if par == 0 else gsem1
            pltpu.async_copy(xprev_h.at[src_d.at[par]], rows_d.at[par], sem)

        def wait_chunk(par):
            sem = gsem0 if par == 0 else gsem1
            pltpu.make_async_copy(
                xprev_h.at[src_d.at[par]], rows_d.at[par], sem).wait()

        def ensure(c, completed):
            def cond(st):
                return st <= c

            def body(st):
                @pl.when(st % 2 == 0)
                def _():
                    wait_chunk(0)

                @pl.when(st % 2 == 1)
                def _():
                    wait_chunk(1)
                nxt = st + 1

                @pl.when((nxt < nch) & (nxt % 2 == 0))
                def _():
                    start_dyn(nxt, 0)

                @pl.when((nxt < nch) & (nxt % 2 == 1))
                def _():
                    start_dyn(nxt, 1)
                return nxt
            return lax.while_loop(cond, body, completed)

        def start_dyn(i, par):
            start(i, par)

        @pl.when(nch > 0)
        def _():
            start(0, 0)

        def row_body(r, completed):
            e0 = extract(r)
            e1 = extract(r + 1)
            c_lo = (e0 - off0) // KE
            c_hi = (e1 - 1 - off0) // KE

            if reg_rows:
                def chunk_body(c, st):
                    completed, regs = st
                    completed = ensure(c, completed)
                    par = c % 2
                    base = off0 + c * KE
                    lo = jnp.maximum(e0, base)
                    hi = jnp.minimum(e1, base + KE)

                    def edge(e, regs):
                        q = e - base
                        w16 = w_d[par, pl.ds((q // 16) * 16, 16)]
                        ws = _bcast_lane(w16, q - (q // 16) * 16)
                        return tuple(
                            regs[j] + rows_d[par, q, pl.ds(j * 16, 16)] * ws
                            for j in range(nvec))
                    regs = lax.fori_loop(lo, hi, edge, regs)
                    return (completed, regs)

                zregs = tuple(jnp.zeros((16,), F32) for _ in range(nvec))
                completed, regs = lax.fori_loop(
                    c_lo, c_hi + 1, chunk_body, (completed, zregs))
                for j in range(nvec):
                    acc[r, pl.ds(j * 16, 16)] = regs[j]
            else:
                def chunk_body(c, completed):
                    completed = ensure(c, completed)
                    par = c % 2
                    base = off0 + c * KE
                    lo = jnp.maximum(e0, base)
                    hi = jnp.minimum(e1, base + KE)

                    def edge(e, _):
                        q = e - base
                        w16 = w_d[par, pl.ds((q // 16) * 16, 16)]
                        ws = _bcast_lane(w16, q - (q // 16) * 16)
                        for j in range(nvec):
                            jsl = pl.ds(j * 16, 16)
                            acc[r, jsl] = (acc[r, jsl]
                                           + rows_d[par, q, jsl] * ws)
                        return 0
                    lax.fori_loop(lo, hi, edge, 0)
                    return completed

                completed = lax.fori_loop(c_lo, c_hi + 1, chunk_body,
                                          completed)
            return completed

        lax.fori_loop(0, rows_pt, row_body, 0)

        for rb in range(n_rblk):
            ra = rb * KE
            if beta != 0.0:
                pltpu.sync_copy(xm2_h.at[pl.ds(row0 + ra, KE)], rows_d.at[0])

            def prow(r, _):
                for j in range(nvec):
                    jsl = pl.ds(j * 16, 16)
                    v = acc[ra + r, jsl] * alpha
                    if beta != 0.0:
                        v = v + rows_d[0, r, jsl] * beta
                    acc[ra + r, jsl] = v
                return 0
            lax.fori_loop(0, KE, prow, 0)
            pltpu.sync_copy(acc.at[pl.ds(ra, KE)],
                            out_h.at[pl.ds(row0 + ra, KE)])

    return step


def _densify(Np, KE):
    rows_pt = Np // NW
    nvec = Np // 16
    mesh = plsc.VectorSubcoreMesh(core_axis_name="c", subcore_axis_name="s")

    @functools.partial(
        pl.kernel,
        out_type=jax.ShapeDtypeStruct((Np, Np), F32),
        mesh=mesh,
        compiler_params=pltpu.CompilerParams(needs_layout_passes=False),
        scratch_types=[
            pltpu.MemorySpace.VMEM((rows_pt, Np), F32),
            pltpu.MemorySpace.VMEM((KE,), I32),
            pltpu.MemorySpace.VMEM((KE,), I32),
            pltpu.MemorySpace.VMEM((KE,), F32),
            pltpu.MemorySpace.VMEM((rows_pt + 16,), I32),
        ],
    )
    def dens(src_h, dst_h, w_h, rp_h, out_h, acc, src_v, dst_v, w_v, rp_v):
        cid = lax.axis_index("c")
        sid = lax.axis_index("s")
        wid = cid * 16 + sid
        row0 = wid * rows_pt
        iot = lax.iota(I32, 16)

        pltpu.sync_copy(rp_h.at[pl.ds(row0, rows_pt + 16)], rp_v)

        def extract(idx):
            base = (idx // 16) * 16
            part = rp_v[pl.ds(base, 16)]
            return jnp.sum(jnp.where(iot == idx - base, part, 0))

        s_lo = extract(0)
        s_hi = extract(rows_pt)
        off0 = (s_lo // 8) * 8
        nch = (s_hi - off0 + KE - 1) // KE

        def zrow(r, _):
            for j in range(nvec):
                acc[r, pl.ds(j * 16, 16)] = jnp.zeros((16,), F32)
            return 0
        lax.fori_loop(0, rows_pt, zrow, 0)

        def chunk(i, _):
            off = off0 + i * KE
            pltpu.sync_copy(src_h.at[pl.ds(off, KE)], src_v)
            pltpu.sync_copy(dst_h.at[pl.ds(off, KE)], dst_v)
            pltpu.sync_copy(w_h.at[pl.ds(off, KE)], w_v)

            def group(g, _):
                sl = pl.ds(g * 16, 16)
                ids = off + g * 16 + iot
                keep = (ids >= s_lo) & (ids < s_hi)
                wm16 = jnp.where(keep, w_v[sl], 0.0)
                dl16 = jnp.clip(dst_v[sl] - row0, 0, rows_pt - 1)
                src16 = src_v[sl]
                for lane in range(16):
                    plsc.addupdate_scatter(acc, [dl16, src16], wm16,
                                           mask=iot == lane)
                return 0
            lax.fori_loop(0, KE // 16, group, 0)
            return 0
        lax.fori_loop(0, nch, chunk, 0)
        pltpu.sync_copy(acc, out_h.at[pl.ds(row0, rows_pt)])

    return dens


def _dense_axpy(Np, Wt, alpha, beta):
    if beta != 0.0:
        def body(l_ref, x_ref, m_ref, o_ref):
            o_ref[...] = (alpha * jnp.dot(l_ref[...], x_ref[...],
                                          preferred_element_type=F32)
                          + beta * m_ref[...])
        n_in = 3
    else:
        def body(l_ref, x_ref, o_ref):
            o_ref[...] = alpha * jnp.dot(l_ref[...], x_ref[...],
                                         preferred_element_type=F32)
        n_in = 2

    return pl.pallas_call(
        body,
        in_specs=[pl.BlockSpec((Np, Np), lambda: (0, 0)),
                  pl.BlockSpec((Np, Wt), lambda: (0, 0))] +
                 ([pl.BlockSpec((Np, Wt), lambda: (0, 0))]
                  if n_in == 3 else []),
        out_specs=pl.BlockSpec((Np, Wt), lambda: (0, 0)),
        out_shape=jax.ShapeDtypeStruct((Np, Wt), F32),
        name=f"dense_axpy_{int(beta != 0)}",
    )


def _cheb_contract(K, Np, Wt, Fout, BN, relu, name):
    nblk = Np // BN

    def body(*refs):
        z_refs, wt_ref, bias_ref, o_ref = refs[:K], refs[K], refs[K + 1], \
            refs[K + 2]
        acc = jnp.zeros((BN, Fout), F32)
        for k in range(K):
            acc = acc + jnp.dot(z_refs[k][...], wt_ref[k],
                                preferred_element_type=F32)
        acc = acc + bias_ref[...]
        if relu:
            acc = jnp.maximum(acc, 0.0)
        o_ref[...] = acc

    return pl.pallas_call(
        body,
        grid=(nblk,),
        in_specs=[pl.BlockSpec((BN, Wt), lambda i: (i, 0))] * K + [
            pl.BlockSpec((K, Wt, Fout), lambda i: (0, 0, 0)),
            pl.BlockSpec((1, Fout), lambda i: (0, 0)),
        ],
        out_specs=pl.BlockSpec((BN, Fout), lambda i: (i, 0)),
        out_shape=jax.ShapeDtypeStruct((Np, Fout), F32),
        name=name,
    )


def _coarsen(M, N, F, BK):
    nblk = N // BK

    def body(b_ref, h_ref, o_ref):
        @pl.when(pl.program_id(0) == 0)
        def _():
            o_ref[...] = jnp.zeros_like(o_ref)
        o_ref[...] += jnp.dot(b_ref[...], h_ref[...],
                              preferred_element_type=F32)

    return pl.pallas_call(
        body,
        grid=(nblk,),
        in_specs=[
            pl.BlockSpec((M, BK), lambda j: (0, j)),
            pl.BlockSpec((BK, F), lambda j: (j, 0)),
        ],
        out_specs=pl.BlockSpec((M, F), lambda j: (0, 0)),
        out_shape=jax.ShapeDtypeStruct((M, F), F32),
        name="coarsen",
    )


def _fc_head(Bn, Kd, Cn):
    def body(h_ref, wt_ref, b_ref, o_ref):
        logits = lax.dot_general(
            h_ref[...], wt_ref[...], (((1,), (1,)), ((), ())),
            preferred_element_type=F32)
        logits = logits + b_ref[...]
        m = jnp.max(logits, axis=1, keepdims=True)
        z = logits - m
        lse = jnp.log(jnp.sum(jnp.exp(z), axis=1, keepdims=True))
        o_ref[...] = z - lse

    return pl.pallas_call(
        body,
        in_specs=[
            pl.BlockSpec((Bn, Kd), lambda: (0, 0)),
            pl.BlockSpec((Cn, Kd), lambda: (0, 0)),
            pl.BlockSpec((1, Cn), lambda: (0, 0)),
        ],
        out_specs=pl.BlockSpec((Bn, Cn), lambda: (0, 0)),
        out_shape=jax.ShapeDtypeStruct((Bn, Cn), F32),
        name="fc_head",
    )


def _sort_edges(src, dst, w, Np, KE):
    E = src.shape[0]
    L = (E + KE - 1) // KE * KE + KE
    pad = L - E
    srcp = jnp.concatenate([src, jnp.zeros((pad,), I32)])
    dstp = jnp.concatenate([dst, jnp.full((pad,), Np, I32)])
    wp = jnp.concatenate([w, jnp.zeros((pad,), F32)])
    order = jnp.argsort(dstp)
    srcs, dsts, ws = srcp[order], dstp[order], wp[order]
    rr = jnp.minimum(jnp.arange(Np + 16, dtype=I32), Np)
    rp = jnp.searchsorted(dsts, rr).astype(I32)
    return srcs, dsts, ws, rp


def _expand_weight(Wr, Fin, Fout):
    K = Wr.shape[0]
    eye = jnp.eye(B, dtype=F32)
    Wm = (Wr[:, :, None, :, None] * eye[None, None, :, None, :])
    return Wm.reshape(K, Fin * B, Fout * B)


def kernel(x, edge_index1, edge_weight1, edge_index2, edge_weight2, b,
           W1, b1, W2, b2, Wfc, bfc):
    Wt1c = 256
    N1p = 10240
    X0 = jnp.transpose(x, (1, 2, 0)).reshape(N1, T * B)
    X0 = jnp.pad(X0, ((0, N1p - N1), (0, Wt1c - T * B)))

    KE1 = 64
    src1, _dst1, w1, rp1 = _sort_edges(edge_index1[0], edge_index1[1],
                                       edge_weight1, N1p, KE1)
    spmv1_a = _build_spmv(N1p, Wt1c, KE1, 1.0, 0.0, "sc_spmv1_first")
    spmv1_b = _build_spmv(N1p, Wt1c, KE1, 2.0, -1.0, "sc_spmv1_rec")

    Z = [X0]
    Z.append(spmv1_a(src1, w1, rp1, X0))
    for _ in range(2, K1):
        Z.append(spmv1_b(src1, w1, rp1, Z[-1], Z[-2]))

    Wm1 = _expand_weight(W1[:, :, 0, :], T, G1)
    Wm1 = jnp.pad(Wm1, ((0, 0), (0, Wt1c - T * B), (0, 0)))
    bias1 = jnp.repeat(b1, B)[None, :]
    H = _cheb_contract(K1, N1p, Wt1c, G1 * B, 512, True, "cheb1")(
        *Z, Wm1, bias1)

    bp = jnp.pad(b, ((0, 0), (0, N1p - N1)))
    Cm = _coarsen(N2, N1p, G1 * B, 1024)(bp, H)

    N2p = 1024
    Wt2c = G1 * B
    Cp = jnp.pad(Cm, ((0, N2p - N2), (0, 0)))

    KE2 = 512
    src2, dst2, w2, rp2 = _sort_edges(edge_index2[0], edge_index2[1],
                                      edge_weight2, N2p, KE2)
    Ld = _densify(N2p, KE2)(src2, dst2, w2, rp2)
    step1 = _dense_axpy(N2p, Wt2c, 1.0, 0.0)
    stepr = _dense_axpy(N2p, Wt2c, 2.0, -1.0)

    Z2 = [Cp]
    Z2.append(step1(Ld, Cp))
    for _ in range(2, K2):
        Z2.append(stepr(Ld, Z2[-1], Z2[-2]))

    Wm2 = _expand_weight(W2[:, 0, :, :], G1, G2)
    bias2 = jnp.repeat(b2, B)[None, :]
    Y = _cheb_contract(K2, N2p, Wt2c, G2 * B, 128, False, "cheb2")(
        *Z2, Wm2, bias2)

    h2d = Y[:N2].reshape(B, N2 * G2)
    return _fc_head(B, N2 * G2, C)(h2d, Wfc.T, bfc[None, :])


# speedup vs baseline: 63.6927x; 1.0143x over previous
import functools

import jax
import jax.numpy as jnp
from jax import lax
from jax.experimental import pallas as pl
from jax.experimental.pallas import tpu as pltpu
from jax.experimental.pallas import tpu_sc as plsc

N1, N2, T, B = 10000, 1000, 15, 16
G1, G2, K1, K2, C = 64, 32, 12, 12, 6

NW = 32
F32 = jnp.float32
I32 = jnp.int32

_GDN = lax.GatherDimensionNumbers(
    offset_dims=(), collapsed_slice_dims=(0,), start_index_map=(0,))


def _bcast_lane(v, e):
    return lax.gather(v, jnp.full((16, 1), e, I32), _GDN, (1,),
                      mode=lax.GatherScatterMode.PROMISE_IN_BOUNDS)


def _build_spmv(Np, Wt, KE, alpha, beta, name):
    rows_pt = Np // NW
    nvec = Wt // 16
    n_rblk = rows_pt // KE
    reg_rows = nvec <= 16

    mesh = plsc.VectorSubcoreMesh(core_axis_name="c", subcore_axis_name="s")

    @functools.partial(
        pl.kernel,
        out_type=jax.ShapeDtypeStruct((Np, Wt), F32),
        mesh=mesh,
        compiler_params=pltpu.CompilerParams(needs_layout_passes=False),
        scratch_types=[
            pltpu.MemorySpace.VMEM((rows_pt, Wt), F32),
            pltpu.MemorySpace.VMEM((2, KE, Wt), F32),
            pltpu.MemorySpace.VMEM((2, KE), I32),
            pltpu.MemorySpace.VMEM((2, KE), F32),
            pltpu.MemorySpace.VMEM((rows_pt + 16,), I32),
            pltpu.SemaphoreType.DMA,
            pltpu.SemaphoreType.DMA,
        ],
        name=name,
    )
    def step(src_h, w_h, rp_h, xprev_h, *rest):
        if beta != 0.0:
            xm2_h, out_h, acc, rows_d, src_d, w_d, rp_v, gsem0, gsem1 = rest
        else:
            xm2_h = None
            out_h, acc, rows_d, src_d, w_d, rp_v, gsem0, gsem1 = rest
        cid = lax.axis_index("c")
        sid = lax.axis_index("s")
        wid = cid * 16 + sid
        row0 = wid * rows_pt
        iot = lax.iota(I32, 16)

        pltpu.sync_copy(rp_h.at[pl.ds(row0, rows_pt + 16)], rp_v)

        def extract(idx):
            base = (idx // 16) * 16
            part = rp_v[pl.ds(base, 16)]
            return jnp.sum(jnp.where(iot == idx - base, part, 0))

        e_lo = extract(0)
        e_hi = extract(rows_pt)
        off0 = (e_lo // KE) * KE
        nch = (e_hi - off0 + KE - 1) // KE

        if not reg_rows:
            def zrow(r, _):
                for j in range(nvec):
                    acc[r, pl.ds(j * 16, 16)] = jnp.zeros((16,), F32)
                return 0
            lax.fori_loop(0, rows_pt, zrow, 0)

        def start(i, par):
            off = off0 + i * KE
            pltpu.sync_copy(src_h.at[pl.ds(off, KE)], src_d.at[par])
            pltpu.sync_copy(w_h.at[pl.ds(off, KE)], w_d.at[par])
            sem = gsem0 if par == 0 else gsem1
            pltpu.async_copy(xprev_h.at[src_d.at[par]], rows_d.at[par], sem)

        def wait_chunk(par):
            sem = gsem0 if par == 0 else gsem1
            pltpu.make_async_copy(
                xprev_h.at[src_d.at[par]], rows_d.at[par], sem).wait()

        def ensure(c, completed):
            def cond(st):
                return st <= c

            def body(st):
                @pl.when(st % 2 == 0)
                def _():
                    wait_chunk(0)

                @pl.when(st % 2 == 1)
                def _():
                    wait_chunk(1)
                nxt = st + 1

                @pl.when((nxt < nch) & (nxt % 2 == 0))
                def _():
                    start_dyn(nxt, 0)

                @pl.when((nxt < nch) & (nxt % 2 == 1))
                def _():
                    start_dyn(nxt, 1)
                return nxt
            return lax.while_loop(cond, body, completed)

        def start_dyn(i, par):
            start(i, par)

        @pl.when(nch > 0)
        def _():
            start(0, 0)

        def row_body(r, completed):
            e0 = extract(r)
            e1 = extract(r + 1)
            c_lo = (e0 - off0) // KE
            c_hi = (e1 - 1 - off0) // KE

            if reg_rows:
                def chunk_body(c, st):
                    completed, regs = st
                    completed = ensure(c, completed)
                    par = c % 2
                    base = off0 + c * KE
                    lo = jnp.maximum(e0, base)
                    hi = jnp.minimum(e1, base + KE)

                    def edge(e, regs):
                        q = e - base
                        w16 = w_d[par, pl.ds((q // 16) * 16, 16)]
                        ws = _bcast_lane(w16, q - (q // 16) * 16)
                        return tuple(
                            regs[j] + rows_d[par, q, pl.ds(j * 16, 16)] * ws
                            for j in range(nvec))
                    regs = lax.fori_loop(lo, hi, edge, regs)
                    return (completed, regs)

                zregs = tuple(jnp.zeros((16,), F32) for _ in range(nvec))
                completed, regs = lax.fori_loop(
                    c_lo, c_hi + 1, chunk_body, (completed, zregs))
                for j in range(nvec):
                    acc[r, pl.ds(j * 16, 16)] = regs[j]
            else:
                def chunk_body(c, completed):
                    completed = ensure(c, completed)
                    par = c % 2
                    base = off0 + c * KE
                    lo = jnp.maximum(e0, base)
                    hi = jnp.minimum(e1, base + KE)

                    def edge(e, _):
                        q = e - base
                        w16 = w_d[par, pl.ds((q // 16) * 16, 16)]
                        ws = _bcast_lane(w16, q - (q // 16) * 16)
                        for j in range(nvec):
                            jsl = pl.ds(j * 16, 16)
                            acc[r, jsl] = (acc[r, jsl]
                                           + rows_d[par, q, jsl] * ws)
                        return 0
                    lax.fori_loop(lo, hi, edge, 0)
                    return completed

                completed = lax.fori_loop(c_lo, c_hi + 1, chunk_body,
                                          completed)
            return completed

        lax.fori_loop(0, rows_pt, row_body, 0)

        for rb in range(n_rblk):
            ra = rb * KE
            if beta != 0.0:
                pltpu.sync_copy(xm2_h.at[pl.ds(row0 + ra, KE)], rows_d.at[0])

            def prow(r, _):
                for j in range(nvec):
                    jsl = pl.ds(j * 16, 16)
                    v = acc[ra + r, jsl] * alpha
                    if beta != 0.0:
                        v = v + rows_d[0, r, jsl] * beta
                    acc[ra + r, jsl] = v
                return 0
            lax.fori_loop(0, KE, prow, 0)
            pltpu.sync_copy(acc.at[pl.ds(ra, KE)],
                            out_h.at[pl.ds(row0 + ra, KE)])

    return step


def _densify(Np, KE):
    rows_pt = Np // NW
    nvec = Np // 16
    mesh = plsc.VectorSubcoreMesh(core_axis_name="c", subcore_axis_name="s")

    @functools.partial(
        pl.kernel,
        out_type=jax.ShapeDtypeStruct((Np, Np), F32),
        mesh=mesh,
        compiler_params=pltpu.CompilerParams(needs_layout_passes=False),
        scratch_types=[
            pltpu.MemorySpace.VMEM((rows_pt, Np), F32),
            pltpu.MemorySpace.VMEM((KE,), I32),
            pltpu.MemorySpace.VMEM((KE,), I32),
            pltpu.MemorySpace.VMEM((KE,), F32),
            pltpu.MemorySpace.VMEM((rows_pt + 16,), I32),
        ],
    )
    def dens(src_h, dst_h, w_h, rp_h, out_h, acc, src_v, dst_v, w_v, rp_v):
        cid = lax.axis_index("c")
        sid = lax.axis_index("s")
        wid = cid * 16 + sid
        row0 = wid * rows_pt
        iot = lax.iota(I32, 16)

        pltpu.sync_copy(rp_h.at[pl.ds(row0, rows_pt + 16)], rp_v)

        def extract(idx):
            base = (idx // 16) * 16
            part = rp_v[pl.ds(base, 16)]
            return jnp.sum(jnp.where(iot == idx - base, part, 0))

        s_lo = extract(0)
        s_hi = extract(rows_pt)
        off0 = (s_lo // 8) * 8
        nch = (s_hi - off0 + KE - 1) // KE

        def zrow(r, _):
            for j in range(nvec):
                acc[r, pl.ds(j * 16, 16)] = jnp.zeros((16,), F32)
            return 0
        lax.fori_loop(0, rows_pt, zrow, 0)

        def chunk(i, _):
            off = off0 + i * KE
            pltpu.sync_copy(src_h.at[pl.ds(off, KE)], src_v)
            pltpu.sync_copy(dst_h.at[pl.ds(off, KE)], dst_v)
            pltpu.sync_copy(w_h.at[pl.ds(off, KE)], w_v)

            def group(g, _):
                sl = pl.ds(g * 16, 16)
                ids = off + g * 16 + iot
                keep = (ids >= s_lo) & (ids < s_hi)
                wm16 = jnp.where(keep, w_v[sl], 0.0)
                dl16 = jnp.clip(dst_v[sl] - row0, 0, rows_pt - 1)
                src16 = src_v[sl]
                for lane in range(16):
                    plsc.addupdate_scatter(acc, [dl16, src16], wm16,
                                           mask=iot == lane)
                return 0
            lax.fori_loop(0, KE // 16, group, 0)
            return 0
        lax.fori_loop(0, nch, chunk, 0)
        pltpu.sync_copy(acc, out_h.at[pl.ds(row0, rows_pt)])

    return dens


def _dense_cheb_chain(K, Np, Wt):
    def body(l_ref, x0_ref, o_ref, xp_ref, xm_ref):
        k = pl.program_id(0)

        @pl.when(k == 0)
        def _():
            xm_ref[...] = x0_ref[...]

        prev = jnp.where(k == 0, x0_ref[...], xp_ref[...])
        y = jnp.dot(l_ref[...], prev, preferred_element_type=F32)
        y = jnp.where(k == 0, y, 2.0 * y - xm_ref[...])
        o_ref[0] = y

        @pl.when(k > 0)
        def _():
            xm_ref[...] = xp_ref[...]
        xp_ref[...] = y

    return pl.pallas_call(
        body,
        grid=(K - 1,),
        in_specs=[pl.BlockSpec((Np, Np), lambda k: (0, 0)),
                  pl.BlockSpec((Np, Wt), lambda k: (0, 0))],
        out_specs=pl.BlockSpec((1, Np, Wt), lambda k: (k, 0, 0)),
        out_shape=jax.ShapeDtypeStruct((K - 1, Np, Wt), F32),
        scratch_shapes=[pltpu.VMEM((Np, Wt), F32),
                        pltpu.VMEM((Np, Wt), F32)],
        name="dense_cheb_chain",
    )


def _cheb_contract(K, Np, Wt, Fout, BN, relu, name):
    nblk = Np // BN

    def body(*refs):
        z_refs, wt_ref, bias_ref, o_ref = refs[:K], refs[K], refs[K + 1], \
            refs[K + 2]
        acc = jnp.zeros((BN, Fout), F32)
        for k in range(K):
            acc = acc + jnp.dot(z_refs[k][...], wt_ref[k],
                                preferred_element_type=F32)
        acc = acc + bias_ref[...]
        if relu:
            acc = jnp.maximum(acc, 0.0)
        o_ref[...] = acc

    return pl.pallas_call(
        body,
        grid=(nblk,),
        in_specs=[pl.BlockSpec((BN, Wt), lambda i: (i, 0))] * K + [
            pl.BlockSpec((K, Wt, Fout), lambda i: (0, 0, 0)),
            pl.BlockSpec((1, Fout), lambda i: (0, 0)),
        ],
        out_specs=pl.BlockSpec((BN, Fout), lambda i: (i, 0)),
        out_shape=jax.ShapeDtypeStruct((Np, Fout), F32),
        name=name,
    )


def _cheb_contract_stacked(K, Np, Wt, Fout, BN, name):
    nblk = Np // BN

    def body(z0_ref, zs_ref, wt_ref, bias_ref, o_ref):
        acc = jnp.dot(z0_ref[...], wt_ref[0], preferred_element_type=F32)
        for k in range(1, K):
            acc = acc + jnp.dot(zs_ref[k - 1], wt_ref[k],
                                preferred_element_type=F32)
        o_ref[...] = acc + bias_ref[...]

    return pl.pallas_call(
        body,
        grid=(nblk,),
        in_specs=[
            pl.BlockSpec((BN, Wt), lambda i: (i, 0)),
            pl.BlockSpec((K - 1, BN, Wt), lambda i: (0, i, 0)),
            pl.BlockSpec((K, Wt, Fout), lambda i: (0, 0, 0)),
            pl.BlockSpec((1, Fout), lambda i: (0, 0)),
        ],
        out_specs=pl.BlockSpec((BN, Fout), lambda i: (i, 0)),
        out_shape=jax.ShapeDtypeStruct((Np, Fout), F32),
        name=name,
    )


def _coarsen(M, N, F, BK):
    nblk = N // BK

    def body(b_ref, h_ref, o_ref):
        @pl.when(pl.program_id(0) == 0)
        def _():
            o_ref[...] = jnp.zeros_like(o_ref)
        o_ref[...] += jnp.dot(b_ref[...], h_ref[...],
                              preferred_element_type=F32)

    return pl.pallas_call(
        body,
        grid=(nblk,),
        in_specs=[
            pl.BlockSpec((M, BK), lambda j: (0, j)),
            pl.BlockSpec((BK, F), lambda j: (j, 0)),
        ],
        out_specs=pl.BlockSpec((M, F), lambda j: (0, 0)),
        out_shape=jax.ShapeDtypeStruct((M, F), F32),
        name="coarsen",
    )


def _fc_head(Bn, Kd, Cn):
    def body(h_ref, wt_ref, b_ref, o_ref):
        logits = lax.dot_general(
            h_ref[...], wt_ref[...], (((1,), (1,)), ((), ())),
            preferred_element_type=F32)
        logits = logits + b_ref[...]
        m = jnp.max(logits, axis=1, keepdims=True)
        z = logits - m
        lse = jnp.log(jnp.sum(jnp.exp(z), axis=1, keepdims=True))
        o_ref[...] = z - lse

    return pl.pallas_call(
        body,
        in_specs=[
            pl.BlockSpec((Bn, Kd), lambda: (0, 0)),
            pl.BlockSpec((Cn, Kd), lambda: (0, 0)),
            pl.BlockSpec((1, Cn), lambda: (0, 0)),
        ],
        out_specs=pl.BlockSpec((Bn, Cn), lambda: (0, 0)),
        out_shape=jax.ShapeDtypeStruct((Bn, Cn), F32),
        name="fc_head",
    )


def _sort_edges(src, dst, w, Np, KE):
    E = src.shape[0]
    L = (E + KE - 1) // KE * KE + KE
    pad = L - E
    srcp = jnp.concatenate([src, jnp.zeros((pad,), I32)])
    dstp = jnp.concatenate([dst, jnp.full((pad,), Np, I32)])
    wp = jnp.concatenate([w, jnp.zeros((pad,), F32)])
    order = jnp.argsort(dstp)
    srcs, dsts, ws = srcp[order], dstp[order], wp[order]
    rr = jnp.minimum(jnp.arange(Np + 16, dtype=I32), Np)
    rp = jnp.searchsorted(dsts, rr).astype(I32)
    return srcs, dsts, ws, rp


def _expand_weight(Wr, Fin, Fout):
    K = Wr.shape[0]
    eye = jnp.eye(B, dtype=F32)
    Wm = (Wr[:, :, None, :, None] * eye[None, None, :, None, :])
    return Wm.reshape(K, Fin * B, Fout * B)


def kernel(x, edge_index1, edge_weight1, edge_index2, edge_weight2, b,
           W1, b1, W2, b2, Wfc, bfc):
    Wt1c = 256
    N1p = 10240
    X0 = jnp.transpose(x, (1, 2, 0)).reshape(N1, T * B)
    X0 = jnp.pad(X0, ((0, N1p - N1), (0, Wt1c - T * B)))

    KE1 = 64
    src1, _dst1, w1, rp1 = _sort_edges(edge_index1[0], edge_index1[1],
                                       edge_weight1, N1p, KE1)
    spmv1_a = _build_spmv(N1p, Wt1c, KE1, 1.0, 0.0, "sc_spmv1_first")
    spmv1_b = _build_spmv(N1p, Wt1c, KE1, 2.0, -1.0, "sc_spmv1_rec")

    Z = [X0]
    Z.append(spmv1_a(src1, w1, rp1, X0))
    for _ in range(2, K1):
        Z.append(spmv1_b(src1, w1, rp1, Z[-1], Z[-2]))

    Wm1 = _expand_weight(W1[:, :, 0, :], T, G1)
    Wm1 = jnp.pad(Wm1, ((0, 0), (0, Wt1c - T * B), (0, 0)))
    bias1 = jnp.repeat(b1, B)[None, :]
    H = _cheb_contract(K1, N1p, Wt1c, G1 * B, 512, True, "cheb1")(
        *Z, Wm1, bias1)

    bp = jnp.pad(b, ((0, 0), (0, N1p - N1)))
    Cm = _coarsen(N2, N1p, G1 * B, 1024)(bp, H)

    N2p = 1024
    Wt2c = G1 * B
    Cp = jnp.pad(Cm, ((0, N2p - N2), (0, 0)))

    KE2 = 512
    src2, dst2, w2, rp2 = _sort_edges(edge_index2[0], edge_index2[1],
                                      edge_weight2, N2p, KE2)
    Ld = _densify(N2p, KE2)(src2, dst2, w2, rp2)
    Z2s = _dense_cheb_chain(K2, N2p, Wt2c)(Ld, Cp)

    Wm2 = _expand_weight(W2[:, 0, :, :], G1, G2)
    bias2 = jnp.repeat(b2, B)[None, :]
    Y = _cheb_contract_stacked(K2, N2p, Wt2c, G2 * B, 128, "cheb2")(
        Cp, Z2s, Wm2, bias2)

    h2d = Y[:N2].reshape(B, N2 * G2)
    return _fc_head(B, N2 * G2, C)(h2d, Wfc.T, bfc[None, :])
